# Initial kernel scaffold; baseline (speedup 1.0000x reference)
import jax
import jax.numpy as jnp
from jax.experimental import pallas as pl


def kernel(x, router_logits, w1, w2):
    raise NotImplementedError("write your pallas kernel here")


# speedup vs baseline: 1.1170x; 1.1170x over previous
import functools

import jax
import jax.numpy as jnp
from jax.experimental import pallas as pl
from jax.experimental.pallas import tpu as pltpu

E = 8
TOPK = 2
BT = 256


def _routing_col(logits, e):
    m = jnp.max(logits, axis=-1, keepdims=True)
    ex = jnp.exp(logits - m)
    gates = ex / jnp.sum(ex, axis=-1, keepdims=True)
    cols = jax.lax.broadcasted_iota(jnp.int32, gates.shape, 1)
    m1 = jnp.max(gates, axis=-1, keepdims=True)
    is_max1 = gates >= m1
    a1 = jnp.min(jnp.where(is_max1, cols, E), axis=-1, keepdims=True)
    mask1 = cols == a1
    g2 = jnp.where(mask1, -jnp.inf, gates)
    m2 = jnp.max(g2, axis=-1, keepdims=True)
    is_max2 = g2 >= m2
    a2 = jnp.min(jnp.where(is_max2, cols, E), axis=-1, keepdims=True)
    mask2 = cols == a2
    denom = m1 + m2
    w_full = (jnp.where(mask1, m1, 0.0) + jnp.where(mask2, m2, 0.0)) / denom
    return jnp.sum(jnp.where(cols == e, w_full, 0.0), axis=-1, keepdims=True)


def _moe_body(logits_ref, x_ref, w1_ref, w2_ref, out_ref, acc_ref):
    e = pl.program_id(0)
    tb = pl.program_id(1)
    x = x_ref[...]
    w1e = w1_ref[0]
    w2e = w2_ref[0]
    h = jax.lax.dot_general(x, w1e, (((1,), (1,)), ((), ())),
                            preferred_element_type=jnp.float32)
    h = (h * jax.nn.sigmoid(h)).astype(jnp.bfloat16)
    y = jax.lax.dot_general(h, w2e, (((1,), (1,)), ((), ())),
                            preferred_element_type=jnp.float32)
    wcol = _routing_col(logits_ref[...], e)
    contrib = y * wcol
    base = tb * BT

    @pl.when(e == 0)
    def _():
        acc_ref[pl.ds(base, BT), :] = contrib

    @pl.when(e > 0)
    def _():
        acc_ref[pl.ds(base, BT), :] += contrib

    @pl.when(e == E - 1)
    def _():
        out_ref[...] = acc_ref[pl.ds(base, BT), :]


@jax.jit
def kernel(x, router_logits, w1, w2):
    T, H = x.shape
    _, I, _ = w1.shape
    x = x.astype(jnp.bfloat16)
    w1 = w1.astype(jnp.bfloat16)
    w2 = w2.astype(jnp.bfloat16)
    nt = T // BT
    grid = (E, nt)
    return pl.pallas_call(
        _moe_body,
        grid=grid,
        in_specs=[
            pl.BlockSpec((BT, E), lambda e, tb: (tb, 0)),
            pl.BlockSpec((BT, H), lambda e, tb: (tb, 0)),
            pl.BlockSpec((1, I, H), lambda e, tb: (e, 0, 0)),
            pl.BlockSpec((1, H, I), lambda e, tb: (e, 0, 0)),
        ],
        out_specs=pl.BlockSpec((BT, H), lambda e, tb: (tb, 0)),
        out_shape=jax.ShapeDtypeStruct((T, H), jnp.float32),
        scratch_shapes=[pltpu.VMEM((T, H), jnp.float32)],
    )(router_logits, x, w1, w2)
